# baseline (speedup 1.0000x reference)
import jax
import jax.numpy as jnp
from jax import lax
from jax.experimental import pallas as pl
from jax.experimental.pallas import tpu as pltpu
from jax.experimental.pallas import tpu_sc as plsc

F = 26
V = 100000
K = 16
B = 16384

NC = 2
NS = 16
NW = NC * NS
ROWS_PER_W = B // NW
BC = 128
STEPS = ROWS_PER_W // BC
GROUPS = BC // 16
IDX_PER_CHUNK = BC * F
IDX_ROWS = IDX_PER_CHUNK // 128
IDX_ROWS_PAD = 32
N_CHUNKS = NW * STEPS


def _fm_body(xe, xw, emb, wtab, out, idx_v, idx_w, ebuf, wbuf, tbuf, ibuf,
             obuf, sem):
    cid = lax.axis_index("c")
    sid = lax.axis_index("s")
    wid = cid * NS + sid

    lanes = lax.iota(jnp.int32, 16)

    def step_fn(step, carry):
        row0 = wid * ROWS_PER_W + step * BC
        chunk = wid * STEPS + step
        pltpu.sync_copy(xe.at[chunk], idx_v)
        pltpu.sync_copy(xw.at[chunk], idx_w)
        copies = []
        for j in range(IDX_ROWS):
            copies.append(pltpu.async_copy(
                emb.at[idx_v.at[j]], ebuf.at[pl.ds(j * 128, 128)], sem))
            copies.append(pltpu.async_copy(
                wtab.at[idx_w.at[j]], wbuf.at[pl.ds(j * 128, 128)], sem))
        for c in copies:
            c.wait()

        def group_fn(g, gcarry):
            goff = g * 16
            lin = wbuf[pl.ds(goff, 16)]
            for f in range(1, F):
                lin = lin + wbuf[pl.ds(f * BC + goff, 16)]
            inter = jnp.zeros((16,), jnp.float32)
            for r16 in range(16):
                rbase = (goff + r16) * F
                e = ebuf[rbase, :]
                s = e
                q = e * e
                for f in range(1, F):
                    e = ebuf[rbase + f, :]
                    s = s + e
                    q = q + e * e
                t = s * s - q
                tot = t[0]
                for i in range(1, 16):
                    tot = tot + t[i]
                inter = jnp.where(lanes == r16, tot, inter)
            z = lin + 0.5 * inter
            obuf[pl.ds(goff, 16)] = 1.0 / (1.0 + jnp.exp(-z))
            return gcarry

        lax.fori_loop(0, GROUPS, group_fn, 0)
        pltpu.sync_copy(obuf, out.at[pl.ds(row0, BC)])
        return carry

    lax.fori_loop(0, STEPS, step_fn, 0)


def kernel(x, emb_tables, weight_tables):
    emb2d = emb_tables.reshape(F * V, K)
    w1d = weight_tables.reshape(F * V)
    fx = x + (jnp.arange(F, dtype=jnp.int32) * V)[None, :]
    pad = IDX_ROWS_PAD * 128 - IDX_PER_CHUNK
    xe = fx.reshape(N_CHUNKS, IDX_PER_CHUNK)
    xe = jnp.pad(xe, ((0, 0), (0, pad))).reshape(N_CHUNKS, IDX_ROWS_PAD, 128)
    xw = fx.T.reshape(F, N_CHUNKS, BC).transpose(1, 0, 2)
    xw = xw.reshape(N_CHUNKS, IDX_PER_CHUNK)
    xw = jnp.pad(xw, ((0, 0), (0, pad))).reshape(N_CHUNKS, IDX_ROWS_PAD, 128)
    mesh = plsc.VectorSubcoreMesh(core_axis_name="c", subcore_axis_name="s")
    fm = pl.kernel(
        _fm_body,
        out_type=jax.ShapeDtypeStruct((B,), jnp.float32),
        mesh=mesh,
        compiler_params=pltpu.CompilerParams(use_tc_tiling_on_sc=False),
        scratch_types=[
            pltpu.VMEM((IDX_ROWS_PAD, 128), jnp.int32),
            pltpu.VMEM((IDX_ROWS_PAD, 128), jnp.int32),
            pltpu.VMEM((IDX_PER_CHUNK, K), jnp.float32),
            pltpu.VMEM((IDX_PER_CHUNK,), jnp.float32),
            pltpu.VMEM((256,), jnp.float32),
            pltpu.VMEM((BC,), jnp.float32),
            pltpu.VMEM((BC,), jnp.float32),
            pltpu.SemaphoreType.DMA,
        ],
    )
    return fm(xe, xw, emb2d, w1d)
